# Initial kernel scaffold; baseline (speedup 1.0000x reference)
import jax
import jax.numpy as jnp
from jax.experimental import pallas as pl


def kernel(x, edge_index, edge_attr, batch, W_emb, b_emb, Wf1, bf1, Ws1, bs1, Wf2, bf2, Ws2, bs2, W1, b1, W2, b2):
    raise NotImplementedError("write your pallas kernel here")


# speedup vs baseline: 1.0279x; 1.0279x over previous
import jax
import jax.numpy as jnp
from jax.experimental import pallas as pl
from jax.experimental.pallas import tpu as pltpu


def _embed_body(x_ref, w_ref, b_ref, o_ref):
    o_ref[...] = jax.nn.relu(
        jnp.dot(x_ref[...], w_ref[...], preferred_element_type=jnp.float32)
        + b_ref[...]
    )


def _embed(x, W, b):
    N, D = x.shape
    H = W.shape[1]
    BLK = 1000
    return pl.pallas_call(
        _embed_body,
        grid=(N // BLK,),
        in_specs=[
            pl.BlockSpec((BLK, D), lambda i: (i, 0)),
            pl.BlockSpec((D, H), lambda i: (0, 0)),
            pl.BlockSpec((H,), lambda i: (0,)),
        ],
        out_specs=pl.BlockSpec((BLK, H), lambda i: (i, 0)),
        out_shape=jax.ShapeDtypeStruct((N, H), jnp.float32),
    )(x, W, b)


def _cgconv(h, edge_index, edge_attr, Wf, bf, Ws, bs):
    src = edge_index[0]
    dst = edge_index[1]
    z = jnp.concatenate([h[dst], h[src], edge_attr], axis=-1)
    m = jax.nn.sigmoid(z @ Wf + bf) * jax.nn.softplus(z @ Ws + bs)
    agg = jnp.zeros_like(h).at[dst].add(m)
    return h + agg


def kernel(x, edge_index, edge_attr, batch, W_emb, b_emb, Wf1, bf1, Ws1, bs1,
           Wf2, bf2, Ws2, bs2, W1, b1, W2, b2):
    h = _embed(x, W_emb, b_emb)
    h = jax.nn.relu(_cgconv(h, edge_index, edge_attr, Wf1, bf1, Ws1, bs1))
    h = jax.nn.relu(_cgconv(h, edge_index, edge_attr, Wf2, bf2, Ws2, bs2))
    G = 128
    sums = jax.ops.segment_sum(h, batch, num_segments=G)
    counts = jax.ops.segment_sum(jnp.ones((h.shape[0], 1), dtype=h.dtype),
                                 batch, num_segments=G)
    pooled = sums / jnp.maximum(counts, 1.0)
    hid = jax.nn.relu(pooled @ W1 + b1)
    out = hid @ W2 + b2
    return out


# speedup vs baseline: 4.3921x; 4.2728x over previous
import functools
import jax
import jax.numpy as jnp
from jax import lax
from jax.experimental import pallas as pl
from jax.experimental.pallas import tpu as pltpu
from jax.experimental.pallas import tpu_sc as plsc

N = 10000
E = 640000
H = 64
G = 128

NC = 2
NS = 16
NW = NC * NS
EPW = E // NW
EB = 80
NBLK = EPW // EB
N_PAD = 10240
ROWS_PER_TILE = N_PAD // NS


def _embed_body(x_ref, we_ref, be_ref, wd_ref, bd_ref, ws_ref,
                h_ref, td_ref, ts_ref):
    h = jax.nn.relu(
        jnp.dot(x_ref[...], we_ref[...], preferred_element_type=jnp.float32)
        + be_ref[...])
    h_ref[...] = h
    td_ref[...] = jnp.dot(h, wd_ref[...], preferred_element_type=jnp.float32) + bd_ref[...]
    ts_ref[...] = jnp.dot(h, ws_ref[...], preferred_element_type=jnp.float32)


def _embed(x, W_emb, b_emb, Wd, bd, Ws):
    BLK = 2000
    return pl.pallas_call(
        _embed_body,
        grid=(N // BLK,),
        in_specs=[
            pl.BlockSpec((BLK, 128), lambda i: (i, 0)),
            pl.BlockSpec((128, H), lambda i: (0, 0)),
            pl.BlockSpec((1, H), lambda i: (0, 0)),
            pl.BlockSpec((H, 128), lambda i: (0, 0)),
            pl.BlockSpec((1, 128), lambda i: (0, 0)),
            pl.BlockSpec((H, 128), lambda i: (0, 0)),
        ],
        out_specs=[
            pl.BlockSpec((BLK, H), lambda i: (i, 0)),
            pl.BlockSpec((BLK, 128), lambda i: (i, 0)),
            pl.BlockSpec((BLK, 128), lambda i: (i, 0)),
        ],
        out_shape=[
            jax.ShapeDtypeStruct((N, H), jnp.float32),
            jax.ShapeDtypeStruct((N, 128), jnp.float32),
            jax.ShapeDtypeStruct((N, 128), jnp.float32),
        ],
    )(x, W_emb, b_emb, Wd, bd, Ws)


def _edgeproj_body(ea_ref, w1_ref, w2_ref, o1_ref, o2_ref):
    ea = ea_ref[...]
    o1_ref[...] = jnp.dot(ea, w1_ref[...], preferred_element_type=jnp.float32)
    o2_ref[...] = jnp.dot(ea, w2_ref[...], preferred_element_type=jnp.float32)


def _edgeproj(edge_attr, We1, We2):
    BLK = 8000
    return pl.pallas_call(
        _edgeproj_body,
        grid=(E // BLK,),
        in_specs=[
            pl.BlockSpec((BLK, 16), lambda i: (i, 0)),
            pl.BlockSpec((16, 128), lambda i: (0, 0)),
            pl.BlockSpec((16, 128), lambda i: (0, 0)),
        ],
        out_specs=[
            pl.BlockSpec((BLK, 128), lambda i: (i, 0)),
            pl.BlockSpec((BLK, 128), lambda i: (i, 0)),
        ],
        out_shape=[
            jax.ShapeDtypeStruct((E, 128), jnp.float32),
            jax.ShapeDtypeStruct((E, 128), jnp.float32),
        ],
    )(edge_attr, We1, We2)


def _mid_body(h_ref, agg_ref, wd_ref, bd_ref, ws_ref, h1_ref, td_ref, ts_ref):
    h1 = jax.nn.relu(h_ref[...] + agg_ref[0, :, :H] + agg_ref[1, :, :H])
    h1_ref[...] = h1
    td_ref[...] = jnp.dot(h1, wd_ref[...], preferred_element_type=jnp.float32) + bd_ref[...]
    ts_ref[...] = jnp.dot(h1, ws_ref[...], preferred_element_type=jnp.float32)


def _mid(h, agg, Wd, bd, Ws):
    BLK = 2000
    return pl.pallas_call(
        _mid_body,
        grid=(N // BLK,),
        in_specs=[
            pl.BlockSpec((BLK, H), lambda i: (i, 0)),
            pl.BlockSpec((NC, BLK, 128), lambda i: (0, i, 0)),
            pl.BlockSpec((H, 128), lambda i: (0, 0)),
            pl.BlockSpec((1, 128), lambda i: (0, 0)),
            pl.BlockSpec((H, 128), lambda i: (0, 0)),
        ],
        out_specs=[
            pl.BlockSpec((BLK, H), lambda i: (i, 0)),
            pl.BlockSpec((BLK, 128), lambda i: (i, 0)),
            pl.BlockSpec((BLK, 128), lambda i: (i, 0)),
        ],
        out_shape=[
            jax.ShapeDtypeStruct((N, H), jnp.float32),
            jax.ShapeDtypeStruct((N, 128), jnp.float32),
            jax.ShapeDtypeStruct((N, 128), jnp.float32),
        ],
    )(h, agg, Wd, bd, Ws)


def _pool_body(h_ref, agg_ref, b_ref, w1_ref, b1_ref, w2_ref, b2_ref,
               out_ref, sums_ref, cnt_ref):
    i = pl.program_id(0)
    nblk = pl.num_programs(0)

    @pl.when(i == 0)
    def _():
        sums_ref[...] = jnp.zeros_like(sums_ref)
        cnt_ref[...] = jnp.zeros_like(cnt_ref)

    h2 = jax.nn.relu(h_ref[...] + agg_ref[0, :, :H] + agg_ref[1, :, :H])
    b = b_ref[0, 0, :]
    gids = lax.broadcasted_iota(jnp.int32, (h2.shape[0], G), 1)
    oh = (b[:, None] == gids).astype(jnp.float32)
    dn = (((0,), (0,)), ((), ()))
    sums_ref[...] += lax.dot_general(oh, h2, dn,
                                     preferred_element_type=jnp.float32)
    ones = jnp.ones((h2.shape[0], G), dtype=jnp.float32)
    cnt_ref[...] += lax.dot_general(oh, ones, dn,
                                    preferred_element_type=jnp.float32)

    @pl.when(i == nblk - 1)
    def _():
        pooled = sums_ref[...] / jnp.maximum(cnt_ref[:, :H], 1.0)
        hid = jax.nn.relu(
            jnp.dot(pooled, w1_ref[...], preferred_element_type=jnp.float32)
            + b1_ref[...])
        out_ref[...] = jnp.dot(hid, w2_ref[...],
                               preferred_element_type=jnp.float32) + b2_ref[...]


def _pool(h, agg, batch_r, W1, b1, W2, b2):
    BLK = 2000
    return pl.pallas_call(
        _pool_body,
        grid=(N // BLK,),
        in_specs=[
            pl.BlockSpec((BLK, H), lambda i: (i, 0)),
            pl.BlockSpec((NC, BLK, 128), lambda i: (0, i, 0)),
            pl.BlockSpec((1, 1, BLK), lambda i: (i, 0, 0)),
            pl.BlockSpec((H, 32), lambda i: (0, 0)),
            pl.BlockSpec((1, 32), lambda i: (0, 0)),
            pl.BlockSpec((32, 2), lambda i: (0, 0)),
            pl.BlockSpec((1, 2), lambda i: (0, 0)),
        ],
        out_specs=pl.BlockSpec((G, 2), lambda i: (0, 0)),
        out_shape=jax.ShapeDtypeStruct((G, 2), jnp.float32),
        scratch_shapes=[
            pltpu.VMEM((G, H), jnp.float32),
            pltpu.VMEM((G, G), jnp.float32),
        ],
    )(h, agg, batch_r, W1, b1, W2, b2)


def _msg_block(dbuf, sbuf, ebuf, mbuf):
    def body(j, _):
        for c in range(4):
            sl = pl.ds(16 * c, 16)
            sh = pl.ds(64 + 16 * c, 16)
            u = dbuf[j, sl] + sbuf[j, sl] + ebuf[j, sl]
            v = dbuf[j, sh] + sbuf[j, sh] + ebuf[j, sh]
            f = 1.0 / (1.0 + jnp.exp(-u))
            e = jnp.exp(-jnp.abs(v))
            t = e / (2.0 + e)
            t2 = t * t
            L = t * (2.0 + t2 * (2.0 / 3.0 + t2 * (2.0 / 5.0 + t2 * (
                2.0 / 7.0 + t2 * (2.0 / 9.0 + t2 * (2.0 / 11.0))))))
            s = jnp.maximum(v, 0.0) + L
            mbuf[j, sl] = f * s
        return 0
    lax.fori_loop(0, EB, body, 0)


def _conv_sc_body(td_hbm, ts_hbm, ep_hbm, dst_hbm, src_hbm, zer_hbm, out_hbm,
                  didx, sidx, dbuf, sbuf, ebuf, mbuf, agg_sh, sem, sem2, sem3):
    cid = lax.axis_index("c")
    sid = lax.axis_index("s")
    wid = cid * NS + sid
    base0 = wid * EPW

    def zmb(j, _):
        for c in range(4, 8):
            mbuf[j, pl.ds(16 * c, 16)] = jnp.zeros((16,), jnp.float32)
        return 0
    lax.fori_loop(0, EB, zmb, 0)

    row0 = sid * ROWS_PER_TILE
    pltpu.sync_copy(zer_hbm.at[pl.ds(row0, ROWS_PER_TILE)],
                    agg_sh.at[pl.ds(row0, ROWS_PER_TILE)])
    plsc.subcore_barrier()

    def step(k, _):
        base = base0 + k * EB
        pltpu.sync_copy(dst_hbm.at[pl.ds(base, EB)], didx)
        pltpu.sync_copy(src_hbm.at[pl.ds(base, EB)], sidx)
        cpd = pltpu.async_copy(td_hbm.at[didx], dbuf, sem)
        cps = pltpu.async_copy(ts_hbm.at[sidx], sbuf, sem2)
        cpe = pltpu.async_copy(ep_hbm.at[pl.ds(base, EB), :], ebuf, sem3)
        cpd.wait()
        cps.wait()
        cpe.wait()
        _msg_block(dbuf, sbuf, ebuf, mbuf)
        pltpu.sync_copy(mbuf, agg_sh.at[didx], add=True)
        return 0

    lax.fori_loop(0, NBLK, step, 0)

    plsc.subcore_barrier()
    pltpu.sync_copy(agg_sh.at[pl.ds(row0, ROWS_PER_TILE)],
                    out_hbm.at[cid, pl.ds(row0, ROWS_PER_TILE)])


def _conv_sc(Td, Ts, EP, dst, src, zeros_nh):
    mesh = plsc.VectorSubcoreMesh(core_axis_name="c", subcore_axis_name="s")
    f = pl.kernel(
        _conv_sc_body,
        out_type=jax.ShapeDtypeStruct((NC, N_PAD, 128), jnp.float32),
        mesh=mesh,
        scratch_types=[
            pltpu.VMEM((EB,), jnp.int32),
            pltpu.VMEM((EB,), jnp.int32),
            pltpu.VMEM((EB, 128), jnp.float32),
            pltpu.VMEM((EB, 128), jnp.float32),
            pltpu.VMEM((EB, 128), jnp.float32),
            pltpu.VMEM((EB, 128), jnp.float32),
            pltpu.VMEM_SHARED((N_PAD, 128), jnp.float32),
            pltpu.SemaphoreType.DMA,
            pltpu.SemaphoreType.DMA,
            pltpu.SemaphoreType.DMA,
        ],
    )
    return f(Td, Ts, EP, dst, src, zeros_nh)


def kernel(x, edge_index, edge_attr, batch, W_emb, b_emb, Wf1, bf1, Ws1, bs1,
           Wf2, bf2, Ws2, bs2, W1, b1, W2, b2):
    src = edge_index[0]
    dst = edge_index[1]

    Wd1 = jnp.concatenate([Wf1[:H], Ws1[:H]], axis=1)
    bd1 = jnp.concatenate([bf1, bs1]).reshape(1, 128)
    Wsr1 = jnp.concatenate([Wf1[H:2 * H], Ws1[H:2 * H]], axis=1)
    We1 = jnp.concatenate([Wf1[2 * H:], Ws1[2 * H:]], axis=1)
    Wd2 = jnp.concatenate([Wf2[:H], Ws2[:H]], axis=1)
    bd2 = jnp.concatenate([bf2, bs2]).reshape(1, 128)
    Wsr2 = jnp.concatenate([Wf2[H:2 * H], Ws2[H:2 * H]], axis=1)
    We2 = jnp.concatenate([Wf2[2 * H:], Ws2[2 * H:]], axis=1)

    zeros_nh = jnp.zeros((N_PAD, 128), jnp.float32)

    h0, Td1, Ts1 = _embed(x, W_emb, b_emb.reshape(1, H), Wd1, bd1, Wsr1)
    EP1, EP2 = _edgeproj(edge_attr, We1, We2)
    agg1 = _conv_sc(Td1, Ts1, EP1, dst, src, zeros_nh)[:, :N]
    h1, Td2, Ts2 = _mid(h0, agg1, Wd2, bd2, Wsr2)
    agg2 = _conv_sc(Td2, Ts2, EP2, dst, src, zeros_nh)[:, :N]
    out = _pool(h1, agg2, batch.reshape(N // 2000, 1, 2000),
                W1, b1.reshape(1, 32), W2, b2.reshape(1, 2))
    return out


# speedup vs baseline: 5.0557x; 1.1511x over previous
import functools
import jax
import jax.numpy as jnp
from jax import lax
from jax.experimental import pallas as pl
from jax.experimental.pallas import tpu as pltpu
from jax.experimental.pallas import tpu_sc as plsc

N = 10000
E = 640000
H = 64
G = 128

NC = 2
NS = 16
NW = NC * NS
EPW = E // NW
EB = 40
NBLK = EPW // EB
N_PAD = 10240
ROWS_PER_TILE = N_PAD // NS


def _embed_body(x_ref, we_ref, be_ref, wd_ref, bd_ref, ws_ref,
                h_ref, td_ref, ts_ref):
    h = jax.nn.relu(
        jnp.dot(x_ref[...], we_ref[...], preferred_element_type=jnp.float32)
        + be_ref[...])
    h_ref[...] = h
    td_ref[...] = jnp.dot(h, wd_ref[...], preferred_element_type=jnp.float32) + bd_ref[...]
    ts_ref[...] = jnp.dot(h, ws_ref[...], preferred_element_type=jnp.float32)


def _embed(x, W_emb, b_emb, Wd, bd, Ws):
    BLK = 2000
    return pl.pallas_call(
        _embed_body,
        grid=(N // BLK,),
        in_specs=[
            pl.BlockSpec((BLK, 128), lambda i: (i, 0)),
            pl.BlockSpec((128, H), lambda i: (0, 0)),
            pl.BlockSpec((1, H), lambda i: (0, 0)),
            pl.BlockSpec((H, 128), lambda i: (0, 0)),
            pl.BlockSpec((1, 128), lambda i: (0, 0)),
            pl.BlockSpec((H, 128), lambda i: (0, 0)),
        ],
        out_specs=[
            pl.BlockSpec((BLK, H), lambda i: (i, 0)),
            pl.BlockSpec((BLK, 128), lambda i: (i, 0)),
            pl.BlockSpec((BLK, 128), lambda i: (i, 0)),
        ],
        out_shape=[
            jax.ShapeDtypeStruct((N, H), jnp.float32),
            jax.ShapeDtypeStruct((N, 128), jnp.float32),
            jax.ShapeDtypeStruct((N, 128), jnp.float32),
        ],
    )(x, W_emb, b_emb, Wd, bd, Ws)


def _edgeproj_body(ea_ref, w1_ref, w2_ref, o1_ref, o2_ref):
    ea = ea_ref[...]
    o1_ref[...] = jnp.dot(ea, w1_ref[...], preferred_element_type=jnp.float32)
    o2_ref[...] = jnp.dot(ea, w2_ref[...], preferred_element_type=jnp.float32)


def _edgeproj(edge_attr, We1, We2):
    BLK = 8000
    return pl.pallas_call(
        _edgeproj_body,
        grid=(E // BLK,),
        in_specs=[
            pl.BlockSpec((BLK, 16), lambda i: (i, 0)),
            pl.BlockSpec((16, 128), lambda i: (0, 0)),
            pl.BlockSpec((16, 128), lambda i: (0, 0)),
        ],
        out_specs=[
            pl.BlockSpec((BLK, 128), lambda i: (i, 0)),
            pl.BlockSpec((BLK, 128), lambda i: (i, 0)),
        ],
        out_shape=[
            jax.ShapeDtypeStruct((E, 128), jnp.float32),
            jax.ShapeDtypeStruct((E, 128), jnp.float32),
        ],
    )(edge_attr, We1, We2)


def _mid_body(h_ref, agg_ref, wd_ref, bd_ref, ws_ref, h1_ref, td_ref, ts_ref):
    h1 = jax.nn.relu(h_ref[...] + agg_ref[0, :, :H] + agg_ref[1, :, :H])
    h1_ref[...] = h1
    td_ref[...] = jnp.dot(h1, wd_ref[...], preferred_element_type=jnp.float32) + bd_ref[...]
    ts_ref[...] = jnp.dot(h1, ws_ref[...], preferred_element_type=jnp.float32)


def _mid(h, agg, Wd, bd, Ws):
    BLK = 2000
    return pl.pallas_call(
        _mid_body,
        grid=(N // BLK,),
        in_specs=[
            pl.BlockSpec((BLK, H), lambda i: (i, 0)),
            pl.BlockSpec((NC, BLK, 128), lambda i: (0, i, 0)),
            pl.BlockSpec((H, 128), lambda i: (0, 0)),
            pl.BlockSpec((1, 128), lambda i: (0, 0)),
            pl.BlockSpec((H, 128), lambda i: (0, 0)),
        ],
        out_specs=[
            pl.BlockSpec((BLK, H), lambda i: (i, 0)),
            pl.BlockSpec((BLK, 128), lambda i: (i, 0)),
            pl.BlockSpec((BLK, 128), lambda i: (i, 0)),
        ],
        out_shape=[
            jax.ShapeDtypeStruct((N, H), jnp.float32),
            jax.ShapeDtypeStruct((N, 128), jnp.float32),
            jax.ShapeDtypeStruct((N, 128), jnp.float32),
        ],
    )(h, agg, Wd, bd, Ws)


def _pool_body(h_ref, agg_ref, b_ref, w1_ref, b1_ref, w2_ref, b2_ref,
               out_ref, sums_ref, cnt_ref):
    i = pl.program_id(0)
    nblk = pl.num_programs(0)

    @pl.when(i == 0)
    def _():
        sums_ref[...] = jnp.zeros_like(sums_ref)
        cnt_ref[...] = jnp.zeros_like(cnt_ref)

    h2 = jax.nn.relu(h_ref[...] + agg_ref[0, :, :H] + agg_ref[1, :, :H])
    b = b_ref[0, 0, :]
    gids = lax.broadcasted_iota(jnp.int32, (h2.shape[0], G), 1)
    oh = (b[:, None] == gids).astype(jnp.float32)
    dn = (((0,), (0,)), ((), ()))
    sums_ref[...] += lax.dot_general(oh, h2, dn,
                                     preferred_element_type=jnp.float32)
    ones = jnp.ones((h2.shape[0], G), dtype=jnp.float32)
    cnt_ref[...] += lax.dot_general(oh, ones, dn,
                                    preferred_element_type=jnp.float32)

    @pl.when(i == nblk - 1)
    def _():
        pooled = sums_ref[...] / jnp.maximum(cnt_ref[:, :H], 1.0)
        hid = jax.nn.relu(
            jnp.dot(pooled, w1_ref[...], preferred_element_type=jnp.float32)
            + b1_ref[...])
        out_ref[...] = jnp.dot(hid, w2_ref[...],
                               preferred_element_type=jnp.float32) + b2_ref[...]


def _pool(h, agg, batch_r, W1, b1, W2, b2):
    BLK = 2000
    return pl.pallas_call(
        _pool_body,
        grid=(N // BLK,),
        in_specs=[
            pl.BlockSpec((BLK, H), lambda i: (i, 0)),
            pl.BlockSpec((NC, BLK, 128), lambda i: (0, i, 0)),
            pl.BlockSpec((1, 1, BLK), lambda i: (i, 0, 0)),
            pl.BlockSpec((H, 32), lambda i: (0, 0)),
            pl.BlockSpec((1, 32), lambda i: (0, 0)),
            pl.BlockSpec((32, 2), lambda i: (0, 0)),
            pl.BlockSpec((1, 2), lambda i: (0, 0)),
        ],
        out_specs=pl.BlockSpec((G, 2), lambda i: (0, 0)),
        out_shape=jax.ShapeDtypeStruct((G, 2), jnp.float32),
        scratch_shapes=[
            pltpu.VMEM((G, H), jnp.float32),
            pltpu.VMEM((G, G), jnp.float32),
        ],
    )(h, agg, batch_r, W1, b1, W2, b2)


def _msg_block(dbuf, sbuf, ebuf, mbuf):
    def body(j, _):
        for c in range(4):
            sl = pl.ds(16 * c, 16)
            sh = pl.ds(64 + 16 * c, 16)
            u = dbuf[j, sl] + sbuf[j, sl] + ebuf[j, sl]
            v = dbuf[j, sh] + sbuf[j, sh] + ebuf[j, sh]
            f = 1.0 / (1.0 + jnp.exp(-u))
            e = jnp.exp(-jnp.abs(v))
            t = e / (2.0 + e)
            t2 = t * t
            L = t * (2.0 + t2 * (2.0 / 3.0 + t2 * (2.0 / 5.0 + t2 * (
                2.0 / 7.0 + t2 * (2.0 / 9.0 + t2 * (2.0 / 11.0))))))
            s = jnp.maximum(v, 0.0) + L
            mbuf[j, sl] = f * s
        return 0
    lax.fori_loop(0, EB, body, 0)


def _conv_sc_body(td_hbm, ts_hbm, ep_hbm, dst_hbm, src_hbm, zer_hbm, out_hbm,
                  didx0, sidx0, dbuf0, sbuf0, ebuf0,
                  didx1, sidx1, dbuf1, sbuf1, ebuf1,
                  mbuf, agg_sh,
                  semd0, sems0, seme0, semd1, sems1, seme1):
    cid = lax.axis_index("c")
    sid = lax.axis_index("s")
    wid = cid * NS + sid
    base0 = wid * EPW
    didx = (didx0, didx1)
    sidx = (sidx0, sidx1)
    dbuf = (dbuf0, dbuf1)
    sbuf = (sbuf0, sbuf1)
    ebuf = (ebuf0, ebuf1)
    semd = (semd0, semd1)
    sems = (sems0, sems1)
    seme = (seme0, seme1)

    def prefetch(base, slot):
        base = jnp.minimum(base, E - EB)
        pltpu.sync_copy(dst_hbm.at[pl.ds(base, EB)], didx[slot])
        pltpu.sync_copy(src_hbm.at[pl.ds(base, EB)], sidx[slot])
        pltpu.async_copy(td_hbm.at[didx[slot]], dbuf[slot], semd[slot])
        pltpu.async_copy(ts_hbm.at[sidx[slot]], sbuf[slot], sems[slot])
        pltpu.async_copy(ep_hbm.at[pl.ds(base, EB), :], ebuf[slot], seme[slot])

    def wait_slot(slot):
        pltpu.make_async_copy(td_hbm.at[didx[slot]], dbuf[slot], semd[slot]).wait()
        pltpu.make_async_copy(ts_hbm.at[sidx[slot]], sbuf[slot], sems[slot]).wait()
        pltpu.make_async_copy(ep_hbm.at[pl.ds(0, EB), :], ebuf[slot], seme[slot]).wait()

    def consume(slot):
        _msg_block(dbuf[slot], sbuf[slot], ebuf[slot], mbuf)
        pltpu.sync_copy(mbuf, agg_sh.at[didx[slot]], add=True)

    def zmb(j, _):
        for c in range(4, 8):
            mbuf[j, pl.ds(16 * c, 16)] = jnp.zeros((16,), jnp.float32)
        return 0
    lax.fori_loop(0, EB, zmb, 0)

    row0 = sid * ROWS_PER_TILE
    pltpu.sync_copy(zer_hbm.at[pl.ds(row0, ROWS_PER_TILE)],
                    agg_sh.at[pl.ds(row0, ROWS_PER_TILE)])
    plsc.subcore_barrier()

    prefetch(base0, 0)

    def two_steps(kk, _):
        b0 = base0 + kk * (2 * EB)
        prefetch(b0 + EB, 1)
        wait_slot(0)
        consume(0)
        prefetch(b0 + 2 * EB, 0)
        wait_slot(1)
        consume(1)
        return 0

    lax.fori_loop(0, NBLK // 2, two_steps, 0)
    wait_slot(0)

    plsc.subcore_barrier()
    pltpu.sync_copy(agg_sh.at[pl.ds(row0, ROWS_PER_TILE)],
                    out_hbm.at[cid, pl.ds(row0, ROWS_PER_TILE)])


def _conv_sc(Td, Ts, EP, dst, src, zeros_nh):
    mesh = plsc.VectorSubcoreMesh(core_axis_name="c", subcore_axis_name="s")
    f = pl.kernel(
        _conv_sc_body,
        out_type=jax.ShapeDtypeStruct((NC, N_PAD, 128), jnp.float32),
        mesh=mesh,
        scratch_types=(
            [pltpu.VMEM((EB,), jnp.int32),
             pltpu.VMEM((EB,), jnp.int32),
             pltpu.VMEM((EB, 128), jnp.float32),
             pltpu.VMEM((EB, 128), jnp.float32),
             pltpu.VMEM((EB, 128), jnp.float32)] * 2
            + [pltpu.VMEM((EB, 128), jnp.float32),
               pltpu.VMEM_SHARED((N_PAD, 128), jnp.float32)]
            + [pltpu.SemaphoreType.DMA] * 6
        ),
    )
    return f(Td, Ts, EP, dst, src, zeros_nh)


def kernel(x, edge_index, edge_attr, batch, W_emb, b_emb, Wf1, bf1, Ws1, bs1,
           Wf2, bf2, Ws2, bs2, W1, b1, W2, b2):
    src = edge_index[0]
    dst = edge_index[1]

    Wd1 = jnp.concatenate([Wf1[:H], Ws1[:H]], axis=1)
    bd1 = jnp.concatenate([bf1, bs1]).reshape(1, 128)
    Wsr1 = jnp.concatenate([Wf1[H:2 * H], Ws1[H:2 * H]], axis=1)
    We1 = jnp.concatenate([Wf1[2 * H:], Ws1[2 * H:]], axis=1)
    Wd2 = jnp.concatenate([Wf2[:H], Ws2[:H]], axis=1)
    bd2 = jnp.concatenate([bf2, bs2]).reshape(1, 128)
    Wsr2 = jnp.concatenate([Wf2[H:2 * H], Ws2[H:2 * H]], axis=1)
    We2 = jnp.concatenate([Wf2[2 * H:], Ws2[2 * H:]], axis=1)

    zeros_nh = jnp.zeros((N_PAD, 128), jnp.float32)

    h0, Td1, Ts1 = _embed(x, W_emb, b_emb.reshape(1, H), Wd1, bd1, Wsr1)
    EP1, EP2 = _edgeproj(edge_attr, We1, We2)
    agg1 = _conv_sc(Td1, Ts1, EP1, dst, src, zeros_nh)[:, :N]
    h1, Td2, Ts2 = _mid(h0, agg1, Wd2, bd2, Wsr2)
    agg2 = _conv_sc(Td2, Ts2, EP2, dst, src, zeros_nh)[:, :N]
    out = _pool(h1, agg2, batch.reshape(N // 2000, 1, 2000),
                W1, b1.reshape(1, 32), W2, b2.reshape(1, 2))
    return out
